# Initial kernel scaffold; baseline (speedup 1.0000x reference)
import jax
import jax.numpy as jnp
from jax.experimental import pallas as pl


def kernel(h, groups):
    raise NotImplementedError("write your pallas kernel here")


# speedup vs baseline: 1.9529x; 1.9529x over previous
import functools

import jax
import jax.numpy as jnp
from jax import lax
from jax.experimental import pallas as pl
from jax.experimental.pallas import tpu as pltpu
from jax.experimental.pallas import tpu_sc as plsc

N = 8192
D = 256
M = 8192
G = 3
OUTW = 2 * D + 4

NW = 32
GP_W = M // NW
CH = 32
NCH = GP_W // CH
IDX = CH * G
SLAB = CH * OUTW + 16


def _mean_body(h_ref, o_ref):
    o_ref[...] = jnp.sum(h_ref[...], axis=0, keepdims=True) * (1.0 / N)


def _col_mean(h):
    return pl.pallas_call(
        _mean_body,
        out_shape=jax.ShapeDtypeStruct((1, D), jnp.float32),
    )(h)


_mesh = plsc.VectorSubcoreMesh(core_axis_name="c", subcore_axis_name="s")


@functools.partial(
    pl.kernel,
    mesh=_mesh,
    out_type=jax.ShapeDtypeStruct((M * OUTW,), jnp.float32),
    scratch_types=[
        pltpu.VMEM((IDX,), jnp.int32),
        pltpu.VMEM((IDX, D), jnp.float32),
        pltpu.VMEM((SLAB,), jnp.float32),
        pltpu.VMEM((D,), jnp.float32),
        pltpu.SemaphoreType.DMA,
    ],
)
def _sc_build(h_hbm, gflat_hbm, hglob_hbm, out_hbm, idx_v, rows_v, slab_v, hg_v, sem):
    cid = lax.axis_index("c")
    sid = lax.axis_index("s")
    wid = sid * 2 + cid
    base_g = wid * GP_W

    pltpu.sync_copy(hglob_hbm, hg_v)

    lane = lax.iota(jnp.int32, 16)
    one0 = jnp.where(lane == 0, jnp.float32(G / 3.0), jnp.float32(0.0))

    def fill_const(g, carry):
        off = g * OUTW
        for c in range(16):
            slab_v[pl.ds(off + D + c * 16, 16)] = hg_v[pl.ds(c * 16, 16)]
        slab_v[pl.ds(off + 2 * D, 16)] = one0
        return carry

    lax.fori_loop(0, CH, fill_const, 0)

    def chunk(k, carry):
        g0 = base_g + k * CH
        pltpu.sync_copy(gflat_hbm.at[pl.ds(g0 * G, IDX)], idx_v)
        pltpu.async_copy(h_hbm.at[idx_v], rows_v, sem).wait()

        def pool(g, c2):
            soff = g * OUTW
            r = g * G
            for c in range(16):
                a = rows_v[r, pl.ds(c * 16, 16)]
                b = rows_v[r + 1, pl.ds(c * 16, 16)]
                d = rows_v[r + 2, pl.ds(c * 16, 16)]
                slab_v[pl.ds(soff + c * 16, 16)] = (a + b + d) * jnp.float32(1.0 / G)
            return c2

        lax.fori_loop(0, CH, pool, 0)
        pltpu.sync_copy(
            slab_v.at[pl.ds(0, CH * OUTW)],
            out_hbm.at[pl.ds(g0 * OUTW, CH * OUTW)],
        )
        return carry

    lax.fori_loop(0, NCH, chunk, 0)


def kernel(h, groups):
    gflat = groups.astype(jnp.int32).reshape(-1)
    hglob = _col_mean(h).reshape(D)
    x_flat = _sc_build(h, gflat, hglob)
    return x_flat.reshape(M, OUTW)


# speedup vs baseline: 2.2037x; 1.1284x over previous
import functools

import jax
import jax.numpy as jnp
from jax import lax
from jax.experimental import pallas as pl
from jax.experimental.pallas import tpu as pltpu
from jax.experimental.pallas import tpu_sc as plsc

N = 8192
D = 256
M = 8192
G = 3
OUTW = 2 * D + 4

NW = 32
GP_W = M // NW
CH = 32
NCH = GP_W // CH
IDX = CH * G
SLAB = CH * OUTW + 16


def _mean_body(h_ref, o_ref):
    o_ref[...] = jnp.sum(h_ref[...], axis=0, keepdims=True) * (1.0 / N)


def _col_mean(h):
    return pl.pallas_call(
        _mean_body,
        out_shape=jax.ShapeDtypeStruct((1, D), jnp.float32),
    )(h)


_mesh = plsc.VectorSubcoreMesh(core_axis_name="c", subcore_axis_name="s")


@functools.partial(
    pl.kernel,
    mesh=_mesh,
    out_type=jax.ShapeDtypeStruct((M * OUTW,), jnp.float32),
    scratch_types=[
        pltpu.VMEM((GP_W * G,), jnp.int32),
        pltpu.VMEM((IDX, D), jnp.float32),
        pltpu.VMEM((IDX, D), jnp.float32),
        pltpu.VMEM((SLAB,), jnp.float32),
        pltpu.VMEM((SLAB,), jnp.float32),
        pltpu.VMEM((D,), jnp.float32),
        pltpu.SemaphoreType.DMA,
        pltpu.SemaphoreType.DMA,
        pltpu.SemaphoreType.DMA,
        pltpu.SemaphoreType.DMA,
    ],
)
def _sc_build(h_hbm, gflat_hbm, hglob_hbm, out_hbm,
              idx_v, rows0, rows1, slab0, slab1, hg_v,
              sg0, sg1, so0, so1):
    cid = lax.axis_index("c")
    sid = lax.axis_index("s")
    wid = sid * 2 + cid
    base_g = wid * GP_W

    rows = (rows0, rows1)
    slabs = (slab0, slab1)
    gsems = (sg0, sg1)
    osems = (so0, so1)

    pltpu.sync_copy(gflat_hbm.at[pl.ds(base_g * G, GP_W * G)], idx_v)
    pltpu.sync_copy(hglob_hbm, hg_v)

    lane = lax.iota(jnp.int32, 16)
    one0 = jnp.where(lane == 0, jnp.float32(G / 3.0), jnp.float32(0.0))

    def fill_const(slab_v):
        def body(g, carry):
            off = g * OUTW
            for c in range(16):
                slab_v[pl.ds(off + D + c * 16, 16)] = hg_v[pl.ds(c * 16, 16)]
            slab_v[pl.ds(off + 2 * D, 16)] = one0
            return carry
        lax.fori_loop(0, CH, body, 0)

    fill_const(slab0)
    fill_const(slab1)

    def start_gather(k):
        b = k % 2
        return pltpu.async_copy(
            h_hbm.at[idx_v.at[pl.ds(k * IDX, IDX)]], rows[b], gsems[b])

    def pool(k):
        b = k % 2
        rows_v, slab_v = rows[b], slabs[b]

        def body(g, carry):
            soff = g * OUTW
            r = g * G
            for c in range(16):
                a = rows_v[r, pl.ds(c * 16, 16)]
                b2 = rows_v[r + 1, pl.ds(c * 16, 16)]
                d2 = rows_v[r + 2, pl.ds(c * 16, 16)]
                slab_v[pl.ds(soff + c * 16, 16)] = (a + b2 + d2) * jnp.float32(1.0 / G)
            return carry

        lax.fori_loop(0, CH, body, 0)

    def start_out(k):
        b = k % 2
        g0 = base_g + k * CH
        return pltpu.async_copy(
            slabs[b].at[pl.ds(0, CH * OUTW)],
            out_hbm.at[pl.ds(g0 * OUTW, CH * OUTW)],
            osems[b])

    ghandles = [None, None]
    ohandles = [None, None]
    ghandles[0] = start_gather(0)
    for k in range(NCH):
        b = k % 2
        if k + 1 < NCH:
            ghandles[1 - b] = start_gather(k + 1)
        ghandles[b].wait()
        if ohandles[b] is not None:
            ohandles[b].wait()
        pool(k)
        ohandles[b] = start_out(k)
    for b in range(2):
        if ohandles[b] is not None:
            ohandles[b].wait()


def kernel(h, groups):
    gflat = groups.astype(jnp.int32).reshape(-1)
    hglob = _col_mean(h).reshape(D)
    x_flat = _sc_build(h, gflat, hglob)
    return x_flat.reshape(M, OUTW)


# speedup vs baseline: 2.7056x; 1.2277x over previous
import functools

import jax
import jax.numpy as jnp
from jax import lax
from jax.experimental import pallas as pl
from jax.experimental.pallas import tpu as pltpu
from jax.experimental.pallas import tpu_sc as plsc

N = 8192
D = 256
M = 8192
G = 3
OUTW = 2 * D + 4
TAILW = D + 4

NW = 32
GP_W = M // NW
CH = 32
NCH = GP_W // CH
IDX = CH * G


def _tmpl_body(h_ref, o_ref):
    mean = jnp.sum(h_ref[...], axis=0, keepdims=True) * (1.0 / N)
    col4 = lax.broadcasted_iota(jnp.int32, (1, 4), 1)
    tail = jnp.where(col4 == 0, jnp.float32(G / 3.0), jnp.float32(0.0))
    o_ref[...] = jnp.concatenate([mean, tail], axis=1)


def _col_mean_tmpl(h):
    return pl.pallas_call(
        _tmpl_body,
        out_shape=jax.ShapeDtypeStruct((1, TAILW), jnp.float32),
    )(h)


_mesh = plsc.VectorSubcoreMesh(core_axis_name="c", subcore_axis_name="s")


@functools.partial(
    pl.kernel,
    mesh=_mesh,
    out_type=jax.ShapeDtypeStruct((M, OUTW), jnp.float32),
    scratch_types=[
        pltpu.VMEM((GP_W * G,), jnp.int32),
        pltpu.VMEM((IDX, D), jnp.float32),
        pltpu.VMEM((IDX, D), jnp.float32),
        pltpu.VMEM((CH, OUTW), jnp.float32),
        pltpu.VMEM((CH, OUTW), jnp.float32),
        pltpu.SemaphoreType.DMA,
        pltpu.SemaphoreType.DMA,
        pltpu.SemaphoreType.DMA,
        pltpu.SemaphoreType.DMA,
    ],
)
def _sc_build(h_hbm, gflat_hbm, tmpl_hbm, out_hbm,
              idx_v, rows0, rows1, slab0, slab1,
              sg0, sg1, so0, so1):
    cid = lax.axis_index("c")
    sid = lax.axis_index("s")
    wid = sid * 2 + cid
    base_g = wid * GP_W

    rows = (rows0, rows1)
    slabs = (slab0, slab1)
    gsems = (sg0, sg1)
    osems = (so0, so1)

    pltpu.sync_copy(gflat_hbm.at[pl.ds(base_g * G, GP_W * G)], idx_v)

    for slab_v in slabs:
        pltpu.sync_copy(tmpl_hbm, slab_v.at[:, pl.ds(D, TAILW)])

    def start_gather(k):
        b = k % 2
        return pltpu.async_copy(
            h_hbm.at[idx_v.at[pl.ds(k * IDX, IDX)]], rows[b], gsems[b])

    def pool(k):
        b = k % 2
        rows_v, slab_v = rows[b], slabs[b]

        def body(g, carry):
            r = g * G
            for c in range(16):
                a = rows_v[r, pl.ds(c * 16, 16)]
                b2 = rows_v[r + 1, pl.ds(c * 16, 16)]
                d2 = rows_v[r + 2, pl.ds(c * 16, 16)]
                slab_v[g, pl.ds(c * 16, 16)] = (a + b2 + d2) * jnp.float32(1.0 / G)
            return carry

        lax.fori_loop(0, CH, body, 0)

    def start_out(k):
        b = k % 2
        g0 = base_g + k * CH
        return pltpu.async_copy(
            slabs[b], out_hbm.at[pl.ds(g0, CH), :], osems[b])

    ghandles = [None, None]
    ohandles = [None, None]
    ghandles[0] = start_gather(0)
    for k in range(NCH):
        b = k % 2
        if k + 1 < NCH:
            ghandles[1 - b] = start_gather(k + 1)
        ghandles[b].wait()
        if ohandles[b] is not None:
            ohandles[b].wait()
        pool(k)
        ohandles[b] = start_out(k)
    for b in range(2):
        if ohandles[b] is not None:
            ohandles[b].wait()


def kernel(h, groups):
    gflat = groups.astype(jnp.int32).reshape(-1)
    tmpl = jnp.broadcast_to(_col_mean_tmpl(h), (CH, TAILW))
    return _sc_build(h, gflat, tmpl)
